# Initial kernel scaffold; baseline (speedup 1.0000x reference)
import jax
import jax.numpy as jnp
from jax.experimental import pallas as pl


def kernel(x, edge_index, batch, W1, b1, W2, b2):
    raise NotImplementedError("write your pallas kernel here")


# speedup vs baseline: 2.9866x; 2.9866x over previous
import functools

import jax
import jax.numpy as jnp
from jax.experimental import pallas as pl


def _mm_body(x_ref, w_ref, o_ref):
    o_ref[...] = jnp.dot(x_ref[...], w_ref[...],
                         preferred_element_type=jnp.float32)


def _matmul(x, w, blk=2000):
    n, k = x.shape
    m = w.shape[1]
    return pl.pallas_call(
        _mm_body,
        grid=(n // blk,),
        in_specs=[pl.BlockSpec((blk, k), lambda i: (i, 0)),
                  pl.BlockSpec((k, m), lambda i: (0, 0))],
        out_specs=pl.BlockSpec((blk, m), lambda i: (i, 0)),
        out_shape=jax.ShapeDtypeStruct((n, m), jnp.float32),
    )(x, w)


def kernel(x, edge_index, batch, W1, b1, W2, b2):
    n = x.shape[0]
    src, dst = edge_index[0], edge_index[1]
    deg = jnp.ones((n,), jnp.float32).at[dst].add(1.0)
    dinv = jax.lax.rsqrt(deg)

    g1 = dinv[:, None] * _matmul(x, W1)
    acc1 = g1.at[dst].add(g1[src])
    h1 = jax.nn.relu(dinv[:, None] * acc1 + b1)

    g2 = dinv[:, None] * _matmul(h1, W2)
    acc2 = g2.at[dst].add(g2[src])
    h2 = dinv[:, None] * acc2 + b2

    out = jax.ops.segment_sum(h2, batch, num_segments=64)
    return out


# speedup vs baseline: 21.4850x; 7.1938x over previous
import functools

import jax
import jax.numpy as jnp
from jax import lax
from jax.experimental import pallas as pl
from jax.experimental.pallas import tpu as pltpu
from jax.experimental.pallas import tpu_sc as plsc

_N = 10000
_E = 320000
_G = 64

_NC = 2
_NS = 16
_NW = _NC * _NS

_CH = 128
_EPW_PAD = -(-(_E // _NW) // _CH) * _CH
_NCHUNK = _EPW_PAD // _CH

_NPAD = 10240
_RPT = _NPAD // _NS
_BLK = 1280
_GRID = _NPAD // _BLK


def _wid(cid, sid):
    return cid * _NS + sid


def _deg_body(dstp, out, dst_v, ones_v, zbuf_v, deg_sh):
    cid = lax.axis_index("c")
    sid = lax.axis_index("s")

    ones16 = jnp.full((16,), 1.0, jnp.float32)
    for j in range(_CH // 16):
        ones_v[pl.ds(j * 16, 16)] = ones16
    zeros16 = jnp.zeros((16,), jnp.float32)

    def zb(i, _):
        zbuf_v[pl.ds(i * 16, 16)] = zeros16
        return ()
    lax.fori_loop(0, _RPT // 16, zb, ())
    pltpu.sync_copy(zbuf_v, deg_sh.at[pl.ds(sid * _RPT, _RPT)])
    plsc.subcore_barrier()

    pltpu.sync_copy(dstp.at[_wid(cid, sid)], dst_v)

    def step(j, _):
        pltpu.sync_copy(ones_v, deg_sh.at[dst_v.at[j]], add=True)
        return ()
    lax.fori_loop(0, _NCHUNK, step, ())
    plsc.subcore_barrier()

    pltpu.sync_copy(deg_sh.at[pl.ds(sid * _RPT, _RPT)],
                    out.at[cid, pl.ds(sid * _RPT, _RPT)])


def _deg_sc(dstp):
    mesh = plsc.VectorSubcoreMesh(core_axis_name="c", subcore_axis_name="s")
    return pl.kernel(
        _deg_body,
        out_type=jax.ShapeDtypeStruct((_NC, _NPAD), jnp.float32),
        mesh=mesh,
        compiler_params=pltpu.CompilerParams(use_tc_tiling_on_sc=False),
        scratch_types=[
            pltpu.VMEM((_NCHUNK, _CH), jnp.int32),
            pltpu.VMEM((_CH,), jnp.float32),
            pltpu.VMEM((_RPT,), jnp.float32),
            pltpu.VMEM_SHARED((_NPAD,), jnp.float32),
        ],
    )(dstp)


def _conv_body(g, srcp, dstp, out, src_v, dst_v, rows_v, zbuf_v, acc_sh):
    cid = lax.axis_index("c")
    sid = lax.axis_index("s")

    zeros16 = jnp.zeros((16,), jnp.float32)

    def zb(i, _):
        for j in range(4):
            zbuf_v[i, pl.ds(j * 16, 16)] = zeros16
        return ()
    lax.fori_loop(0, 64, zb, ())
    for r in range(_RPT // 64):
        pltpu.sync_copy(zbuf_v, acc_sh.at[pl.ds(sid * _RPT + r * 64, 64)])
    plsc.subcore_barrier()

    w = _wid(cid, sid)
    pltpu.sync_copy(srcp.at[w], src_v)
    pltpu.sync_copy(dstp.at[w], dst_v)

    def step(j, _):
        pltpu.sync_copy(g.at[src_v.at[j]], rows_v)
        pltpu.sync_copy(rows_v, acc_sh.at[dst_v.at[j]], add=True)
        return ()
    lax.fori_loop(0, _NCHUNK, step, ())
    plsc.subcore_barrier()

    pltpu.sync_copy(acc_sh.at[pl.ds(sid * _RPT, _RPT)],
                    out.at[cid, pl.ds(sid * _RPT, _RPT)])


def _conv_sc(g, srcp, dstp):
    mesh = plsc.VectorSubcoreMesh(core_axis_name="c", subcore_axis_name="s")
    return pl.kernel(
        _conv_body,
        out_type=jax.ShapeDtypeStruct((_NC, _NPAD, 64), jnp.float32),
        mesh=mesh,
        compiler_params=pltpu.CompilerParams(use_tc_tiling_on_sc=False),
        scratch_types=[
            pltpu.VMEM((_NCHUNK, _CH), jnp.int32),
            pltpu.VMEM((_NCHUNK, _CH), jnp.int32),
            pltpu.VMEM((_CH, 64), jnp.float32),
            pltpu.VMEM((64, 64), jnp.float32),
            pltpu.VMEM_SHARED((_NPAD, 64), jnp.float32),
        ],
    )(g, srcp, dstp)


def _dinv(dpt_ref):
    deg = dpt_ref[:, 0:1] + dpt_ref[:, 1:2] + 1.0
    return lax.rsqrt(deg)


def _mm1_body(x_ref, w_ref, dpt_ref, g_ref):
    h = jnp.dot(x_ref[...], w_ref[...], preferred_element_type=jnp.float32)
    g_ref[...] = _dinv(dpt_ref) * h


def _mid_body(ap_ref, g1_ref, dpt_ref, b1_ref, w2_ref, g2_ref):
    dinv = _dinv(dpt_ref)
    acc = ap_ref[0] + ap_ref[1] + g1_ref[...]
    h1 = jnp.maximum(dinv * acc + b1_ref[...], 0.0)
    g2_ref[...] = dinv * jnp.dot(h1, w2_ref[...],
                                 preferred_element_type=jnp.float32)


def _pool_body(ap_ref, g2_ref, dpt_ref, b2_ref, bat_ref, out_ref):
    i = pl.program_id(0)
    dinv = _dinv(dpt_ref)
    h2 = dinv * (ap_ref[0] + ap_ref[1] + g2_ref[...]) + b2_ref[...]
    ids = jax.lax.broadcasted_iota(jnp.int32, (_G, _BLK), 0)
    oht = (ids == bat_ref[0]).astype(jnp.float32)
    part = jnp.dot(oht, h2, preferred_element_type=jnp.float32)

    @pl.when(i == 0)
    def _():
        out_ref[...] = part

    @pl.when(i > 0)
    def _():
        out_ref[...] += part


def _mm1_tc(xp, W1, dpt):
    return pl.pallas_call(
        _mm1_body,
        grid=(_GRID,),
        in_specs=[pl.BlockSpec((_BLK, 128), lambda i: (i, 0)),
                  pl.BlockSpec((128, 64), lambda i: (0, 0)),
                  pl.BlockSpec((_BLK, _NC), lambda i: (i, 0))],
        out_specs=pl.BlockSpec((_BLK, 64), lambda i: (i, 0)),
        out_shape=jax.ShapeDtypeStruct((_NPAD, 64), jnp.float32),
    )(xp, W1, dpt)


def _mid_tc(ap, g1, dpt, b1, W2):
    return pl.pallas_call(
        _mid_body,
        grid=(_GRID,),
        in_specs=[pl.BlockSpec((_NC, _BLK, 64), lambda i: (0, i, 0)),
                  pl.BlockSpec((_BLK, 64), lambda i: (i, 0)),
                  pl.BlockSpec((_BLK, _NC), lambda i: (i, 0)),
                  pl.BlockSpec((1, 64), lambda i: (0, 0)),
                  pl.BlockSpec((64, 64), lambda i: (0, 0))],
        out_specs=pl.BlockSpec((_BLK, 64), lambda i: (i, 0)),
        out_shape=jax.ShapeDtypeStruct((_NPAD, 64), jnp.float32),
    )(ap, g1, dpt, b1, W2)


def _pool_tc(ap, g2, dpt, b2, bat3):
    return pl.pallas_call(
        _pool_body,
        grid=(_GRID,),
        in_specs=[pl.BlockSpec((_NC, _BLK, 64), lambda i: (0, i, 0)),
                  pl.BlockSpec((_BLK, 64), lambda i: (i, 0)),
                  pl.BlockSpec((_BLK, _NC), lambda i: (i, 0)),
                  pl.BlockSpec((1, 64), lambda i: (0, 0)),
                  pl.BlockSpec((1, 1, _BLK), lambda i: (i, 0, 0))],
        out_specs=pl.BlockSpec((_G, 64), lambda i: (0, 0)),
        out_shape=jax.ShapeDtypeStruct((_G, 64), jnp.float32),
    )(ap, g2, dpt, b2, bat3)


def kernel(x, edge_index, batch, W1, b1, W2, b2):
    src, dst = edge_index[0], edge_index[1]
    epw = _E // _NW
    pad = _EPW_PAD - epw
    srcp = jnp.pad(src.reshape(_NW, epw), ((0, 0), (0, pad))
                   ).reshape(_NW, _NCHUNK, _CH)
    dstp = jnp.pad(dst.reshape(_NW, epw), ((0, 0), (0, pad)),
                   constant_values=_N).reshape(_NW, _NCHUNK, _CH)

    xp = jnp.pad(x, ((0, _NPAD - _N), (0, 0)))
    bat3 = jnp.pad(batch, (0, _NPAD - _N),
                   constant_values=_G).reshape(_GRID, 1, _BLK)

    dp = _deg_sc(dstp)
    dpt = dp.T

    g1 = _mm1_tc(xp, W1, dpt)
    ap1 = _conv_sc(g1, srcp, dstp)
    g2 = _mid_tc(ap1, g1, dpt, b1.reshape(1, 64), W2)
    ap2 = _conv_sc(g2, srcp, dstp)
    out = _pool_tc(ap2, g2, dpt, b2.reshape(1, 64), bat3)
    return out
